# baseline (speedup 1.0000x reference)
import functools

import jax
import jax.numpy as jnp
from jax import lax
from jax.experimental import pallas as pl
from jax.experimental.pallas import tpu as pltpu
from jax.experimental.pallas import tpu_sc as plsc

BATCH = 4096
SEQ = 200
EMBED = 64
HIDDEN = 256
NUM_CLASSES = 10
OUT_PAD = 128

_NC = 2
_NS = 16
_NW = _NC * _NS
_RPW = BATCH // _NW

_CHUNKS = ((0, 128), (128, 72))
_NSLOT = 4
_AHEAD = 3


def _pool_body(x_hbm, e_hbm, out_hbm, xv, bv0, bv1, bv2, bv3, outv,
               sem0, sem1, sem2, sem3):
    wid = lax.axis_index("s") * _NC + lax.axis_index("c")
    base = wid * _RPW
    pltpu.sync_copy(x_hbm.at[pl.ds(base, _RPW)], xv)

    bufs = (bv0, bv1, bv2, bv3)
    sems = (sem0, sem1, sem2, sem3)
    inv_seq = 1.0 / SEQ

    def copies(row, slot):
        return [
            pltpu.make_async_copy(
                e_hbm.at[xv.at[row, pl.ds(off, cnt)]],
                bufs[slot].at[pl.ds(off, cnt)],
                sems[slot])
            for (off, cnt) in _CHUNKS
        ]

    def accumulate(row, slot):
        bs = bufs[slot]

        def acc_step(i, accs):
            s = 4 * i
            new = list(accs)
            for u in range(4):
                for j in range(4):
                    new[j] = new[j] + bs[s + u, pl.ds(16 * j, 16)]
            return tuple(new)

        zero = jnp.zeros((16,), jnp.float32)
        accs = lax.fori_loop(0, SEQ // 4, acc_step, (zero,) * 4)
        for j in range(4):
            outv[row, pl.ds(16 * j, 16)] = accs[j] * inv_seq

    for r in range(_AHEAD):
        for c in copies(r, r % _NSLOT):
            c.start()

    def group_step(i, carry):
        for j in range(_NSLOT):
            row = _NSLOT * i + j
            for c in copies(row, j):
                c.wait()
            nrow = row + _AHEAD
            nslot = (j + _AHEAD) % _NSLOT

            @pl.when(nrow < _RPW)
            def _():
                for c in copies(nrow, nslot):
                    c.start()

            accumulate(row, j)
        return carry

    lax.fori_loop(0, _RPW // _NSLOT, group_step, 0)
    pltpu.sync_copy(outv, out_hbm.at[pl.ds(base, _RPW)])


VOCAB = 100000
_NTC = VOCAB // 128
_TAIL = VOCAB - _NTC * 128
_TCQ, _TCR = divmod(_NTC, _NW)


def _conv_body(et_hbm, tail_hbm, out_hbm, iv0, iv1, ov0, ov1, semi0, semi1,
               semo0, semo1):
    wid = lax.axis_index("s") * _NC + lax.axis_index("c")
    start = wid * _TCQ + jnp.minimum(wid, _TCR)
    count = _TCQ + jnp.where(wid < _TCR, 1, 0)

    ibufs, obufs = (iv0, iv1), (ov0, ov1)
    isems, osems = (semi0, semi1), (semo0, semo1)

    def stage(tc, slot):
        return pltpu.make_async_copy(
            et_hbm.at[:, pl.ds(tc * 128, 128)], ibufs[slot], isems[slot])

    def flush(tc, slot):
        return pltpu.make_async_copy(
            obufs[slot], out_hbm.at[pl.ds(tc * 64, 64)], osems[slot])

    evecs = [lax.iota(jnp.int32, 16) + (16 * j) for j in range(4)]

    def transpose_block(slot):
        ib, ob = ibufs[slot], obufs[slot]

        def col_step(i, carry):
            c0 = 4 * i
            k0 = 2 * i
            for u in range(4):
                cs = jnp.full((16,), c0 + u, jnp.int32)
                k = k0 + u // 2
                half = (u % 2) * EMBED
                for j in range(4):
                    v = plsc.load_gather(ib, [evecs[j], cs])
                    ob[k, pl.ds(half + 16 * j, 16)] = v
            return carry

        lax.fori_loop(0, 32, col_step, 0)

    stage(start, 0).start()

    def step(i, carry):
        for s in range(2):
            tc = 2 * i + s
            @pl.when(tc < count)
            def _():
                stage(start + tc, s).wait()
                nt = tc + 1
                @pl.when(nt < count)
                def _():
                    stage(start + nt, 1 - s).start()
                @pl.when(tc >= 2)
                def _():
                    flush(start + tc - 2, s).wait()
                transpose_block(s)
                flush(start + tc, s).start()
        return carry

    lax.fori_loop(0, (_TCQ + 2) // 2, step, 0)

    @pl.when(wid < _TCR)
    def _():
        n = _TCQ + 1
        flush(start + n - 2, (n - 2) % 2).wait()
        flush(start + n - 1, (n - 1) % 2).wait()

    @pl.when(wid >= _TCR)
    def _():
        n = _TCQ
        flush(start + n - 2, (n - 2) % 2).wait()
        flush(start + n - 1, (n - 1) % 2).wait()

    @pl.when(wid == _NW - 1)
    def _():
        pltpu.sync_copy(tail_hbm, iv0.at[pl.ds(0, _TAIL // 2)])
        pltpu.sync_copy(iv0.at[pl.ds(0, _TAIL // 2)],
                        out_hbm.at[pl.ds(_NTC * 64, _TAIL // 2)])


@jax.jit
def _convert_table(E):
    mesh = plsc.VectorSubcoreMesh(core_axis_name="c", subcore_axis_name="s")
    out = pl.kernel(
        _conv_body,
        out_type=jax.ShapeDtypeStruct((VOCAB // 2, 128), jnp.float32),
        mesh=mesh,
        scratch_types=[
            pltpu.VMEM((EMBED, 128), jnp.float32),
            pltpu.VMEM((EMBED, 128), jnp.float32),
            pltpu.VMEM((EMBED, 128), jnp.float32),
            pltpu.VMEM((EMBED, 128), jnp.float32),
            pltpu.SemaphoreType.DMA,
            pltpu.SemaphoreType.DMA,
            pltpu.SemaphoreType.DMA,
            pltpu.SemaphoreType.DMA,
        ],
        compiler_params=pltpu.CompilerParams(
            use_tc_tiling_on_sc=True, needs_layout_passes=False),
    )(E.T, lax.slice(E, (VOCAB - _TAIL, 0), (VOCAB, EMBED))
           .reshape(_TAIL // 2, 2 * EMBED))
    return out.reshape(VOCAB, EMBED)


@jax.jit
def _pool(x, E):
    mesh = plsc.VectorSubcoreMesh(core_axis_name="c", subcore_axis_name="s")
    return pl.kernel(
        _pool_body,
        out_type=jax.ShapeDtypeStruct((BATCH, EMBED), jnp.float32),
        mesh=mesh,
        scratch_types=[
            pltpu.VMEM((_RPW, SEQ), jnp.int32),
            pltpu.VMEM((SEQ, EMBED), jnp.float32),
            pltpu.VMEM((SEQ, EMBED), jnp.float32),
            pltpu.VMEM((SEQ, EMBED), jnp.float32),
            pltpu.VMEM((SEQ, EMBED), jnp.float32),
            pltpu.VMEM((_RPW, EMBED), jnp.float32),
            pltpu.SemaphoreType.DMA,
            pltpu.SemaphoreType.DMA,
            pltpu.SemaphoreType.DMA,
            pltpu.SemaphoreType.DMA,
        ],
        compiler_params=pltpu.CompilerParams(use_tc_tiling_on_sc=False),
    )(x, E)


def _mlp_body(p1_ref, p2_ref, p3_ref, w1a_ref, w1b_ref, w1c_ref, b1_ref,
              w2_ref, b2_ref, o_ref):
    h = jnp.dot(p1_ref[...], w1a_ref[...], preferred_element_type=jnp.float32)
    h += jnp.dot(p2_ref[...], w1b_ref[...], preferred_element_type=jnp.float32)
    h += jnp.dot(p3_ref[...], w1c_ref[...], preferred_element_type=jnp.float32)
    h = jnp.maximum(h + b1_ref[...], 0.0)
    o_ref[...] = (
        jnp.dot(h, w2_ref[...], preferred_element_type=jnp.float32)
        + b2_ref[...])


@jax.jit
def _mlp(p1, p2, p3, W1a, W1b, W1c, b1, W2p, b2p):
    blk = 256
    grid = BATCH // blk
    full = lambda i: (0, 0)
    return pl.pallas_call(
        _mlp_body,
        grid=(grid,),
        in_specs=[
            pl.BlockSpec((blk, EMBED), lambda i: (i, 0)),
            pl.BlockSpec((blk, EMBED), lambda i: (i, 0)),
            pl.BlockSpec((blk, EMBED), lambda i: (i, 0)),
            pl.BlockSpec((EMBED, HIDDEN), full),
            pl.BlockSpec((EMBED, HIDDEN), full),
            pl.BlockSpec((EMBED, HIDDEN), full),
            pl.BlockSpec((1, HIDDEN), full),
            pl.BlockSpec((HIDDEN, OUT_PAD), full),
            pl.BlockSpec((1, OUT_PAD), full),
        ],
        out_specs=pl.BlockSpec((blk, OUT_PAD), lambda i: (i, 0)),
        out_shape=jax.ShapeDtypeStruct((BATCH, OUT_PAD), jnp.float32),
    )(p1, p2, p3, W1a, W1b, W1c, b1, W2p, b2p)


def kernel(x, E1, E2, E3, W1, b1, W2, b2):
    xi = x.astype(jnp.int32)
    p1 = _pool(xi, _convert_table(E1))
    p2 = _pool(xi, _convert_table(E2))
    p3 = _pool(xi, _convert_table(E3))
    W2p = jnp.pad(W2, ((0, 0), (0, OUT_PAD - NUM_CLASSES)))
    b2p = jnp.pad(b2, (0, OUT_PAD - NUM_CLASSES)).reshape(1, OUT_PAD)
    out = _mlp(p1, p2, p3, W1[:EMBED], W1[EMBED:2 * EMBED], W1[2 * EMBED:],
               b1.reshape(1, HIDDEN), W2p, b2p)
    return out[:, :NUM_CLASSES]


# speedup vs baseline: 1.7032x; 1.7032x over previous
import functools

import jax
import jax.numpy as jnp
from jax import lax
from jax.experimental import pallas as pl
from jax.experimental.pallas import tpu as pltpu
from jax.experimental.pallas import tpu_sc as plsc

BATCH = 4096
SEQ = 200
EMBED = 64
HIDDEN = 256
NUM_CLASSES = 10
OUT_PAD = 128

_NC = 2
_NS = 16
_NW = _NC * _NS
_RPW = BATCH // _NW

_CHUNKS = ((0, 128), (128, 72))
_NSLOT = 4
_AHEAD = 3


def _pool_body(x_hbm, e_hbm, out_hbm, xv, bv0, bv1, bv2, bv3, outv,
               sem0, sem1, sem2, sem3):
    wid = lax.axis_index("s") * _NC + lax.axis_index("c")
    base = wid * _RPW
    pltpu.sync_copy(x_hbm.at[pl.ds(base, _RPW)], xv)

    bufs = (bv0, bv1, bv2, bv3)
    sems = (sem0, sem1, sem2, sem3)
    inv_seq = 1.0 / SEQ

    def copies(row, slot):
        return [
            pltpu.make_async_copy(
                e_hbm.at[xv.at[row, pl.ds(off, cnt)]],
                bufs[slot].at[pl.ds(off, cnt)],
                sems[slot])
            for (off, cnt) in _CHUNKS
        ]

    def accumulate(row, slot):
        bs = bufs[slot]

        def acc_step(i, accs):
            s = 4 * i
            new = list(accs)
            for u in range(4):
                for j in range(4):
                    new[j] = new[j] + bs[s + u, pl.ds(16 * j, 16)]
            return tuple(new)

        zero = jnp.zeros((16,), jnp.float32)
        accs = lax.fori_loop(0, SEQ // 4, acc_step, (zero,) * 4)
        for j in range(4):
            outv[row, pl.ds(16 * j, 16)] = accs[j] * inv_seq

    for r in range(_AHEAD):
        for c in copies(r, r % _NSLOT):
            c.start()

    def group_step(i, carry):
        for j in range(_NSLOT):
            row = _NSLOT * i + j
            for c in copies(row, j):
                c.wait()
            nrow = row + _AHEAD
            nslot = (j + _AHEAD) % _NSLOT

            @pl.when(nrow < _RPW)
            def _():
                for c in copies(nrow, nslot):
                    c.start()

            accumulate(row, j)
        return carry

    lax.fori_loop(0, _RPW // _NSLOT, group_step, 0)
    pltpu.sync_copy(outv, out_hbm.at[pl.ds(base, _RPW)])


VOCAB = 100000
_NTC = VOCAB // 128
_TAIL = VOCAB - _NTC * 128
_TCQ, _TCR = divmod(_NTC, _NW)


def _conv_body(et_hbm, tail_hbm, out_hbm, iv0, iv1, ov0, ov1, semi0, semi1,
               semo0, semo1):
    wid = lax.axis_index("s") * _NC + lax.axis_index("c")
    start = wid * _TCQ + jnp.minimum(wid, _TCR)
    count = _TCQ + jnp.where(wid < _TCR, 1, 0)

    ibufs, obufs = (iv0, iv1), (ov0, ov1)
    isems, osems = (semi0, semi1), (semo0, semo1)

    def stage(tc, slot):
        return pltpu.make_async_copy(
            et_hbm.at[:, pl.ds(tc * 128, 128)], ibufs[slot], isems[slot])

    def flush(tc, slot):
        return pltpu.make_async_copy(
            obufs[slot], out_hbm.at[pl.ds(tc * 64, 64)], osems[slot])

    lanes = lax.iota(jnp.int32, 16)
    evecs = [lanes + (16 * j) for j in range(4)]

    def transpose_block(slot):
        ib, ob = ibufs[slot], obufs[slot]

        def diag_step(d, carry):
            perm = lax.rem(lanes + d, 16)
            permh = lax.shift_right_logical(perm, 1)
            permb = lax.shift_left(lax.rem(perm, 2), 6)
            for c0 in range(0, 128, 16):
                cvec = perm + c0
                kvec = permh + (c0 // 2)
                for j in range(4):
                    v = plsc.load_gather(ib, [evecs[j], cvec])
                    plsc.store_scatter(ob, [kvec, permb + evecs[j]], v)
            return carry

        lax.fori_loop(0, 16, diag_step, 0)

    stage(start, 0).start()

    def step(i, carry):
        for s in range(2):
            tc = 2 * i + s
            @pl.when(tc < count)
            def _():
                stage(start + tc, s).wait()
                nt = tc + 1
                @pl.when(nt < count)
                def _():
                    stage(start + nt, 1 - s).start()
                @pl.when(tc >= 2)
                def _():
                    flush(start + tc - 2, s).wait()
                transpose_block(s)
                flush(start + tc, s).start()
        return carry

    lax.fori_loop(0, (_TCQ + 2) // 2, step, 0)

    @pl.when(wid < _TCR)
    def _():
        n = _TCQ + 1
        flush(start + n - 2, (n - 2) % 2).wait()
        flush(start + n - 1, (n - 1) % 2).wait()

    @pl.when(wid >= _TCR)
    def _():
        n = _TCQ
        flush(start + n - 2, (n - 2) % 2).wait()
        flush(start + n - 1, (n - 1) % 2).wait()

    @pl.when(wid == _NW - 1)
    def _():
        pltpu.sync_copy(tail_hbm, iv0.at[pl.ds(0, _TAIL // 2)])
        pltpu.sync_copy(iv0.at[pl.ds(0, _TAIL // 2)],
                        out_hbm.at[pl.ds(_NTC * 64, _TAIL // 2)])


@jax.jit
def _convert_table(E):
    mesh = plsc.VectorSubcoreMesh(core_axis_name="c", subcore_axis_name="s")
    out = pl.kernel(
        _conv_body,
        out_type=jax.ShapeDtypeStruct((VOCAB // 2, 128), jnp.float32),
        mesh=mesh,
        scratch_types=[
            pltpu.VMEM((EMBED, 128), jnp.float32),
            pltpu.VMEM((EMBED, 128), jnp.float32),
            pltpu.VMEM((EMBED, 128), jnp.float32),
            pltpu.VMEM((EMBED, 128), jnp.float32),
            pltpu.SemaphoreType.DMA,
            pltpu.SemaphoreType.DMA,
            pltpu.SemaphoreType.DMA,
            pltpu.SemaphoreType.DMA,
        ],
        compiler_params=pltpu.CompilerParams(
            use_tc_tiling_on_sc=True, needs_layout_passes=False),
    )(E.T, lax.slice(E, (VOCAB - _TAIL, 0), (VOCAB, EMBED))
           .reshape(_TAIL // 2, 2 * EMBED))
    return out.reshape(VOCAB, EMBED)


@jax.jit
def _pool(x, E):
    mesh = plsc.VectorSubcoreMesh(core_axis_name="c", subcore_axis_name="s")
    return pl.kernel(
        _pool_body,
        out_type=jax.ShapeDtypeStruct((BATCH, EMBED), jnp.float32),
        mesh=mesh,
        scratch_types=[
            pltpu.VMEM((_RPW, SEQ), jnp.int32),
            pltpu.VMEM((SEQ, EMBED), jnp.float32),
            pltpu.VMEM((SEQ, EMBED), jnp.float32),
            pltpu.VMEM((SEQ, EMBED), jnp.float32),
            pltpu.VMEM((SEQ, EMBED), jnp.float32),
            pltpu.VMEM((_RPW, EMBED), jnp.float32),
            pltpu.SemaphoreType.DMA,
            pltpu.SemaphoreType.DMA,
            pltpu.SemaphoreType.DMA,
            pltpu.SemaphoreType.DMA,
        ],
        compiler_params=pltpu.CompilerParams(use_tc_tiling_on_sc=False),
    )(x, E)


def _mlp_body(p1_ref, p2_ref, p3_ref, w1a_ref, w1b_ref, w1c_ref, b1_ref,
              w2_ref, b2_ref, o_ref):
    h = jnp.dot(p1_ref[...], w1a_ref[...], preferred_element_type=jnp.float32)
    h += jnp.dot(p2_ref[...], w1b_ref[...], preferred_element_type=jnp.float32)
    h += jnp.dot(p3_ref[...], w1c_ref[...], preferred_element_type=jnp.float32)
    h = jnp.maximum(h + b1_ref[...], 0.0)
    o_ref[...] = (
        jnp.dot(h, w2_ref[...], preferred_element_type=jnp.float32)
        + b2_ref[...])


@jax.jit
def _mlp(p1, p2, p3, W1a, W1b, W1c, b1, W2p, b2p):
    blk = 256
    grid = BATCH // blk
    full = lambda i: (0, 0)
    return pl.pallas_call(
        _mlp_body,
        grid=(grid,),
        in_specs=[
            pl.BlockSpec((blk, EMBED), lambda i: (i, 0)),
            pl.BlockSpec((blk, EMBED), lambda i: (i, 0)),
            pl.BlockSpec((blk, EMBED), lambda i: (i, 0)),
            pl.BlockSpec((EMBED, HIDDEN), full),
            pl.BlockSpec((EMBED, HIDDEN), full),
            pl.BlockSpec((EMBED, HIDDEN), full),
            pl.BlockSpec((1, HIDDEN), full),
            pl.BlockSpec((HIDDEN, OUT_PAD), full),
            pl.BlockSpec((1, OUT_PAD), full),
        ],
        out_specs=pl.BlockSpec((blk, OUT_PAD), lambda i: (i, 0)),
        out_shape=jax.ShapeDtypeStruct((BATCH, OUT_PAD), jnp.float32),
    )(p1, p2, p3, W1a, W1b, W1c, b1, W2p, b2p)


def kernel(x, E1, E2, E3, W1, b1, W2, b2):
    xi = x.astype(jnp.int32)
    p1 = _pool(xi, _convert_table(E1))
    p2 = _pool(xi, _convert_table(E2))
    p3 = _pool(xi, _convert_table(E3))
    W2p = jnp.pad(W2, ((0, 0), (0, OUT_PAD - NUM_CLASSES)))
    b2p = jnp.pad(b2, (0, OUT_PAD - NUM_CLASSES)).reshape(1, OUT_PAD)
    out = _mlp(p1, p2, p3, W1[:EMBED], W1[EMBED:2 * EMBED], W1[2 * EMBED:],
               b1.reshape(1, HIDDEN), W2p, b2p)
    return out[:, :NUM_CLASSES]


# speedup vs baseline: 1.9986x; 1.1734x over previous
import functools

import jax
import jax.numpy as jnp
from jax import lax
from jax.experimental import pallas as pl
from jax.experimental.pallas import tpu as pltpu
from jax.experimental.pallas import tpu_sc as plsc

BATCH = 4096
SEQ = 200
EMBED = 64
HIDDEN = 256
NUM_CLASSES = 10
OUT_PAD = 128

_NC = 2
_NS = 16
_NW = _NC * _NS
_RPW = BATCH // _NW

_CHUNKS = ((0, 128), (128, 72))
_NSLOT = 4
_AHEAD = 3


def _pool_body(x_hbm, e_hbm, out_hbm, xv, bv0, bv1, bv2, bv3, outv,
               sem0, sem1, sem2, sem3):
    wid = lax.axis_index("s") * _NC + lax.axis_index("c")
    base = wid * _RPW
    pltpu.sync_copy(x_hbm.at[pl.ds(base, _RPW)], xv)

    bufs = (bv0, bv1, bv2, bv3)
    sems = (sem0, sem1, sem2, sem3)
    inv_seq = 1.0 / SEQ

    def copies(row, slot):
        return [
            pltpu.make_async_copy(
                e_hbm.at[xv.at[row, pl.ds(off, cnt)]],
                bufs[slot].at[pl.ds(off, cnt)],
                sems[slot])
            for (off, cnt) in _CHUNKS
        ]

    def accumulate(row, slot):
        bs = bufs[slot]

        def acc_step(i, accs):
            s = 4 * i
            new = list(accs)
            for u in range(4):
                for j in range(4):
                    new[j] = new[j] + bs[s + u, pl.ds(16 * j, 16)]
            return tuple(new)

        zero = jnp.zeros((16,), jnp.float32)
        accs = lax.fori_loop(0, SEQ // 4, acc_step, (zero,) * 4)
        for j in range(4):
            outv[row, pl.ds(16 * j, 16)] = accs[j] * inv_seq

    for r in range(_AHEAD):
        for c in copies(r, r % _NSLOT):
            c.start()

    def group_step(i, carry):
        for j in range(_NSLOT):
            row = _NSLOT * i + j
            for c in copies(row, j):
                c.wait()
            nrow = row + _AHEAD
            nslot = (j + _AHEAD) % _NSLOT

            @pl.when(nrow < _RPW)
            def _():
                for c in copies(nrow, nslot):
                    c.start()

            accumulate(row, j)
        return carry

    lax.fori_loop(0, _RPW // _NSLOT, group_step, 0)
    pltpu.sync_copy(outv, out_hbm.at[pl.ds(base, _RPW)])


VOCAB = 100000
_NTC = VOCAB // 128
_TAIL = VOCAB - _NTC * 128
_TCQ, _TCR = divmod(_NTC, _NW)


def _conv_body(et_hbm, tail_hbm, out_hbm, iv0, iv1, ov0, ov1, semi0, semi1,
               semo0, semo1):
    wid = lax.axis_index("s") * _NC + lax.axis_index("c")
    start = wid * _TCQ + jnp.minimum(wid, _TCR)
    count = _TCQ + jnp.where(wid < _TCR, 1, 0)

    ibufs, obufs = (iv0, iv1), (ov0, ov1)
    isems, osems = (semi0, semi1), (semo0, semo1)

    def stage(tc, slot):
        return pltpu.make_async_copy(
            et_hbm.at[:, pl.ds(tc * 128, 128)], ibufs[slot], isems[slot])

    def flush(tc, slot):
        return pltpu.make_async_copy(
            obufs[slot], out_hbm.at[pl.ds(tc * 64, 64)], osems[slot])

    lanes = lax.iota(jnp.int32, 16)
    evecs = [lanes + (16 * j) for j in range(4)]

    def transpose_block(slot):
        ib, ob = ibufs[slot], obufs[slot]

        def diag_step(d, carry):
            perm = lax.rem(lanes + d, 16)
            permh = lax.shift_right_logical(perm, 1)
            permb = lax.shift_left(lax.rem(perm, 2), 6)
            lvecs = [permb + evecs[j] for j in range(4)]
            for c0 in range(0, 128, 32):
                cva = perm + c0
                cvb = perm + (c0 + 16)
                kva = permh + (c0 // 2)
                kvb = permh + (c0 // 2 + 8)
                vs = [plsc.load_gather(ib, [evecs[j], cva]) for j in range(4)]
                vs += [plsc.load_gather(ib, [evecs[j], cvb]) for j in range(4)]
                for j in range(4):
                    plsc.store_scatter(ob, [kva, lvecs[j]], vs[j])
                for j in range(4):
                    plsc.store_scatter(ob, [kvb, lvecs[j]], vs[4 + j])
            return carry

        lax.fori_loop(0, 16, diag_step, 0)

    stage(start, 0).start()

    def step(i, carry):
        for s in range(2):
            tc = 2 * i + s
            @pl.when(tc < count)
            def _():
                stage(start + tc, s).wait()
                nt = tc + 1
                @pl.when(nt < count)
                def _():
                    stage(start + nt, 1 - s).start()
                @pl.when(tc >= 2)
                def _():
                    flush(start + tc - 2, s).wait()
                transpose_block(s)
                flush(start + tc, s).start()
        return carry

    lax.fori_loop(0, (_TCQ + 2) // 2, step, 0)

    @pl.when(wid < _TCR)
    def _():
        n = _TCQ + 1
        flush(start + n - 2, (n - 2) % 2).wait()
        flush(start + n - 1, (n - 1) % 2).wait()

    @pl.when(wid >= _TCR)
    def _():
        n = _TCQ
        flush(start + n - 2, (n - 2) % 2).wait()
        flush(start + n - 1, (n - 1) % 2).wait()

    @pl.when(wid == _NW - 1)
    def _():
        pltpu.sync_copy(tail_hbm, iv0.at[pl.ds(0, _TAIL // 2)])
        pltpu.sync_copy(iv0.at[pl.ds(0, _TAIL // 2)],
                        out_hbm.at[pl.ds(_NTC * 64, _TAIL // 2)])


@jax.jit
def _convert_table(E):
    mesh = plsc.VectorSubcoreMesh(core_axis_name="c", subcore_axis_name="s")
    out = pl.kernel(
        _conv_body,
        out_type=jax.ShapeDtypeStruct((VOCAB // 2, 128), jnp.float32),
        mesh=mesh,
        scratch_types=[
            pltpu.VMEM((EMBED, 128), jnp.float32),
            pltpu.VMEM((EMBED, 128), jnp.float32),
            pltpu.VMEM((EMBED, 128), jnp.float32),
            pltpu.VMEM((EMBED, 128), jnp.float32),
            pltpu.SemaphoreType.DMA,
            pltpu.SemaphoreType.DMA,
            pltpu.SemaphoreType.DMA,
            pltpu.SemaphoreType.DMA,
        ],
        compiler_params=pltpu.CompilerParams(
            use_tc_tiling_on_sc=True, needs_layout_passes=False),
    )(E.T, lax.slice(E, (VOCAB - _TAIL, 0), (VOCAB, EMBED))
           .reshape(_TAIL // 2, 2 * EMBED))
    return out.reshape(VOCAB, EMBED)


@jax.jit
def _pool(x, E):
    mesh = plsc.VectorSubcoreMesh(core_axis_name="c", subcore_axis_name="s")
    return pl.kernel(
        _pool_body,
        out_type=jax.ShapeDtypeStruct((BATCH, EMBED), jnp.float32),
        mesh=mesh,
        scratch_types=[
            pltpu.VMEM((_RPW, SEQ), jnp.int32),
            pltpu.VMEM((SEQ, EMBED), jnp.float32),
            pltpu.VMEM((SEQ, EMBED), jnp.float32),
            pltpu.VMEM((SEQ, EMBED), jnp.float32),
            pltpu.VMEM((SEQ, EMBED), jnp.float32),
            pltpu.VMEM((_RPW, EMBED), jnp.float32),
            pltpu.SemaphoreType.DMA,
            pltpu.SemaphoreType.DMA,
            pltpu.SemaphoreType.DMA,
            pltpu.SemaphoreType.DMA,
        ],
        compiler_params=pltpu.CompilerParams(use_tc_tiling_on_sc=False),
    )(x, E)


def _mlp_body(p1_ref, p2_ref, p3_ref, w1a_ref, w1b_ref, w1c_ref, b1_ref,
              w2_ref, b2_ref, o_ref):
    h = jnp.dot(p1_ref[...], w1a_ref[...], preferred_element_type=jnp.float32)
    h += jnp.dot(p2_ref[...], w1b_ref[...], preferred_element_type=jnp.float32)
    h += jnp.dot(p3_ref[...], w1c_ref[...], preferred_element_type=jnp.float32)
    h = jnp.maximum(h + b1_ref[...], 0.0)
    o_ref[...] = (
        jnp.dot(h, w2_ref[...], preferred_element_type=jnp.float32)
        + b2_ref[...])


@jax.jit
def _mlp(p1, p2, p3, W1a, W1b, W1c, b1, W2p, b2p):
    blk = 256
    grid = BATCH // blk
    full = lambda i: (0, 0)
    return pl.pallas_call(
        _mlp_body,
        grid=(grid,),
        in_specs=[
            pl.BlockSpec((blk, EMBED), lambda i: (i, 0)),
            pl.BlockSpec((blk, EMBED), lambda i: (i, 0)),
            pl.BlockSpec((blk, EMBED), lambda i: (i, 0)),
            pl.BlockSpec((EMBED, HIDDEN), full),
            pl.BlockSpec((EMBED, HIDDEN), full),
            pl.BlockSpec((EMBED, HIDDEN), full),
            pl.BlockSpec((1, HIDDEN), full),
            pl.BlockSpec((HIDDEN, OUT_PAD), full),
            pl.BlockSpec((1, OUT_PAD), full),
        ],
        out_specs=pl.BlockSpec((blk, OUT_PAD), lambda i: (i, 0)),
        out_shape=jax.ShapeDtypeStruct((BATCH, OUT_PAD), jnp.float32),
    )(p1, p2, p3, W1a, W1b, W1c, b1, W2p, b2p)


def kernel(x, E1, E2, E3, W1, b1, W2, b2):
    xi = x.astype(jnp.int32)
    p1 = _pool(xi, _convert_table(E1))
    p2 = _pool(xi, _convert_table(E2))
    p3 = _pool(xi, _convert_table(E3))
    W2p = jnp.pad(W2, ((0, 0), (0, OUT_PAD - NUM_CLASSES)))
    b2p = jnp.pad(b2, (0, OUT_PAD - NUM_CLASSES)).reshape(1, OUT_PAD)
    out = _mlp(p1, p2, p3, W1[:EMBED], W1[EMBED:2 * EMBED], W1[2 * EMBED:],
               b1.reshape(1, HIDDEN), W2p, b2p)
    return out[:, :NUM_CLASSES]
